# Initial kernel scaffold; baseline (speedup 1.0000x reference)
import jax
import jax.numpy as jnp
from jax.experimental import pallas as pl


def kernel(x, edge_index, W, att_src, att_dst, bias):
    raise NotImplementedError("write your pallas kernel here")


# speedup vs baseline: 36.7759x; 36.7759x over previous
import functools

import jax
import jax.numpy as jnp
from jax import lax
from jax.experimental import pallas as pl
from jax.experimental.pallas import tpu as pltpu
from jax.experimental.pallas import tpu_sc as plsc

N = 10000
D = 128
H = 8
C = 16
L = 16
NC = 2
NS = 16
NW = NC * NS
CH = 128
NACC = 10112
IROWS = NACC // NS

_SC_MESH = plsc.VectorSubcoreMesh(core_axis_name="c", subcore_axis_name="s")


def _dense1_body(x_ref, w_ref, a_ref, h_ref, s_ref, d_ref, e_ref):
    hb = jnp.dot(x_ref[...], w_ref[...], preferred_element_type=jnp.float32)
    p = jnp.dot(hb, a_ref[...], preferred_element_type=jnp.float32)
    s16 = p[:, :16]
    d16 = p[:, 16:32]
    a = jnp.minimum(s16 + d16, 60.0)
    a = jnp.maximum(a, 0.2 * a)
    h_ref[...] = hb
    s_ref[...] = s16
    d_ref[...] = d16
    e_ref[...] = jnp.exp(a)


def _dense2_body(p0_ref, p1_ref, e_ref, rd_ref, sw_ref):
    dt = p0_ref[...] + p1_ref[...] + e_ref[...]
    rd = 1.0 / (dt + 1e-16)
    lane = lax.broadcasted_iota(jnp.int32, rd.shape, 1)
    rd = jnp.where(lane < H, rd, 0.0)
    rd_ref[...] = rd
    sw_ref[...] = e_ref[...] * rd


def _dense3_body(p0_ref, p1_ref, h_ref, sw_ref, ex8_ref, b_ref, o_ref):
    swl = jnp.dot(sw_ref[...], ex8_ref[...], preferred_element_type=jnp.float32)
    t = p0_ref[...] + p1_ref[...] + h_ref[...] * swl + b_ref[...]
    o_ref[...] = jnp.where(t > 0.0, t, jnp.exp(jnp.minimum(t, 0.0)) - 1.0)


def _sc_pass1_body(src_hbm, dst_hbm, s16_hbm, d16_hbm, z16_hbm,
                   ex_hbm, den_hbm,
                   sidx, didx, gs, gd, exb, sem_s, sem_d, den_s):
    c = lax.axis_index("c")
    s = lax.axis_index("s")
    w = s * NC + c
    ept = src_hbm.shape[0] // NW
    nch = ept // CH

    pltpu.sync_copy(z16_hbm.at[pl.ds(s * IROWS, IROWS), :],
                    den_s.at[pl.ds(s * IROWS, IROWS), :])
    plsc.subcore_barrier()

    def chunk(k, carry):
        base = w * ept + k * CH
        pltpu.sync_copy(src_hbm.at[pl.ds(base, CH)], sidx)
        pltpu.sync_copy(dst_hbm.at[pl.ds(base, CH)], didx)
        cps = pltpu.async_copy(s16_hbm.at[sidx], gs, sem_s)
        cpd = pltpu.async_copy(d16_hbm.at[didx], gd, sem_d)
        cps.wait()
        cpd.wait()

        def edge(e, carry2):
            a = gs[e, :] + gd[e, :]
            a = jnp.minimum(a, 60.0)
            a = jnp.maximum(a, 0.2 * a)
            exb[e, :] = jnp.exp(a)
            return carry2

        lax.fori_loop(0, CH, edge, 0)
        pltpu.sync_copy(exb, ex_hbm.at[pl.ds(base, CH), :])
        pltpu.sync_copy(exb, den_s.at[didx], add=True)
        return carry

    lax.fori_loop(0, nch, chunk, 0)
    plsc.subcore_barrier()
    pltpu.sync_copy(den_s.at[pl.ds(s * IROWS, IROWS), :],
                    den_hbm.at[c, pl.ds(s * IROWS, IROWS), :])


def _sc_pass2_body(src_hbm, dst_hbm, h_hbm, rd_hbm, ex_hbm, z128_hbm,
                   out_hbm,
                   sidx, didx, hrows, rdr, exr, msg, sem_h, sem_r, out_s):
    c = lax.axis_index("c")
    s = lax.axis_index("s")
    w = s * NC + c
    ept = src_hbm.shape[0] // NW
    nch = ept // CH

    pltpu.sync_copy(z128_hbm.at[pl.ds(s * IROWS, IROWS), :],
                    out_s.at[pl.ds(s * IROWS, IROWS), :])
    plsc.subcore_barrier()

    def chunk(k, carry):
        base = w * ept + k * CH
        pltpu.sync_copy(src_hbm.at[pl.ds(base, CH)], sidx)
        pltpu.sync_copy(dst_hbm.at[pl.ds(base, CH)], didx)
        cph = pltpu.async_copy(h_hbm.at[sidx], hrows, sem_h)
        cpr = pltpu.async_copy(rd_hbm.at[didx], rdr, sem_r)
        pltpu.sync_copy(ex_hbm.at[pl.ds(base, CH), :], exr)
        cph.wait()
        cpr.wait()

        def edge(e, carry2):
            wv = rdr[e, :] * exr[e, :]
            for h in range(H):
                sp = jnp.broadcast_to(wv[h], (L,))
                msg[e, pl.ds(h * L, L)] = hrows[e, pl.ds(h * L, L)] * sp
            return carry2

        lax.fori_loop(0, CH, edge, 0)
        pltpu.sync_copy(msg, out_s.at[didx], add=True)
        return carry

    lax.fori_loop(0, nch, chunk, 0)
    plsc.subcore_barrier()
    pltpu.sync_copy(out_s.at[pl.ds(s * IROWS, IROWS), :],
                    out_hbm.at[c, pl.ds(s * IROWS, IROWS), :])


def _expand_att(att):
    eye = jnp.eye(H, dtype=att.dtype)
    m = (att[:, :, None] * eye[:, None, :]).reshape(D, H)
    return jnp.pad(m, ((0, 0), (0, 16 - H)))


@jax.jit
def kernel(x, edge_index, W, att_src, att_dst, bias):
    n = x.shape[0]
    e = edge_index.shape[1]
    epad = ((e + NW * CH - 1) // (NW * CH)) * (NW * CH)

    a_all = jnp.concatenate([_expand_att(att_src), _expand_att(att_dst)], axis=1)

    blk = 400
    grid = n // blk
    h, s16, d16, e16 = pl.pallas_call(
        _dense1_body,
        grid=(grid,),
        in_specs=[
            pl.BlockSpec((blk, D), lambda i: (i, 0)),
            pl.BlockSpec((D, D), lambda i: (0, 0)),
            pl.BlockSpec((D, 32), lambda i: (0, 0)),
        ],
        out_specs=[
            pl.BlockSpec((blk, D), lambda i: (i, 0)),
            pl.BlockSpec((blk, 16), lambda i: (i, 0)),
            pl.BlockSpec((blk, 16), lambda i: (i, 0)),
            pl.BlockSpec((blk, 16), lambda i: (i, 0)),
        ],
        out_shape=[
            jax.ShapeDtypeStruct((n, D), jnp.float32),
            jax.ShapeDtypeStruct((n, 16), jnp.float32),
            jax.ShapeDtypeStruct((n, 16), jnp.float32),
            jax.ShapeDtypeStruct((n, 16), jnp.float32),
        ],
    )(x, W, a_all)

    pad = epad - e
    srcp = jnp.concatenate([edge_index[0], jnp.full((pad,), n, jnp.int32)])
    dstp = jnp.concatenate([edge_index[1], jnp.full((pad,), n, jnp.int32)])
    zrow16 = jnp.zeros((1, 16), jnp.float32)
    s16p = jnp.concatenate([s16, zrow16])
    d16p = jnp.concatenate([d16, zrow16])
    z16 = jnp.zeros((NACC, 16), jnp.float32)

    pass1 = functools.partial(
        pl.kernel,
        out_type=[
            jax.ShapeDtypeStruct((epad, 16), jnp.float32),
            jax.ShapeDtypeStruct((NC, NACC, 16), jnp.float32),
        ],
        mesh=_SC_MESH,
        scratch_types=[
            pltpu.VMEM((CH,), jnp.int32),
            pltpu.VMEM((CH,), jnp.int32),
            pltpu.VMEM((CH, 16), jnp.float32),
            pltpu.VMEM((CH, 16), jnp.float32),
            pltpu.VMEM((CH, 16), jnp.float32),
            pltpu.SemaphoreType.DMA,
            pltpu.SemaphoreType.DMA,
            pltpu.VMEM_SHARED((NACC, 16), jnp.float32),
        ],
        compiler_params=pltpu.CompilerParams(use_tc_tiling_on_sc=False),
    )(_sc_pass1_body)
    exh, den = pass1(srcp, dstp, s16p, d16p, z16)
    den = den[:, :n, :]

    rd16, selfw = pl.pallas_call(
        _dense2_body,
        grid=(grid,),
        in_specs=[
            pl.BlockSpec((blk, 16), lambda i: (i, 0)),
            pl.BlockSpec((blk, 16), lambda i: (i, 0)),
            pl.BlockSpec((blk, 16), lambda i: (i, 0)),
        ],
        out_specs=[
            pl.BlockSpec((blk, 16), lambda i: (i, 0)),
            pl.BlockSpec((blk, 16), lambda i: (i, 0)),
        ],
        out_shape=[
            jax.ShapeDtypeStruct((n, 16), jnp.float32),
            jax.ShapeDtypeStruct((n, 16), jnp.float32),
        ],
    )(den[0], den[1], e16)

    hp = jnp.concatenate([h, jnp.zeros((1, D), jnp.float32)])
    rdp = jnp.concatenate([rd16, zrow16])
    z128 = jnp.zeros((NACC, D), jnp.float32)

    pass2 = functools.partial(
        pl.kernel,
        out_type=[jax.ShapeDtypeStruct((NC, NACC, D), jnp.float32)],
        mesh=_SC_MESH,
        scratch_types=[
            pltpu.VMEM((CH,), jnp.int32),
            pltpu.VMEM((CH,), jnp.int32),
            pltpu.VMEM((CH, D), jnp.float32),
            pltpu.VMEM((CH, 16), jnp.float32),
            pltpu.VMEM((CH, 16), jnp.float32),
            pltpu.VMEM((CH, D), jnp.float32),
            pltpu.SemaphoreType.DMA,
            pltpu.SemaphoreType.DMA,
            pltpu.VMEM_SHARED((NACC, D), jnp.float32),
        ],
        compiler_params=pltpu.CompilerParams(use_tc_tiling_on_sc=False),
    )(_sc_pass2_body)
    (outp,) = pass2(srcp, dstp, hp, rdp, exh, z128)
    outp = outp[:, :n, :]

    ex8 = (jnp.arange(16)[:, None] == (jnp.arange(D)[None, :] // C)).astype(jnp.float32)
    out = pl.pallas_call(
        _dense3_body,
        grid=(grid,),
        in_specs=[
            pl.BlockSpec((blk, D), lambda i: (i, 0)),
            pl.BlockSpec((blk, D), lambda i: (i, 0)),
            pl.BlockSpec((blk, D), lambda i: (i, 0)),
            pl.BlockSpec((blk, 16), lambda i: (i, 0)),
            pl.BlockSpec((16, D), lambda i: (0, 0)),
            pl.BlockSpec((1, D), lambda i: (0, 0)),
        ],
        out_specs=pl.BlockSpec((blk, D), lambda i: (i, 0)),
        out_shape=jax.ShapeDtypeStruct((n, D), jnp.float32),
    )(outp[0], outp[1], h, selfw, ex8, bias.reshape(1, D))
    return out
